# Initial kernel scaffold; baseline (speedup 1.0000x reference)
import jax
import jax.numpy as jnp
from jax.experimental import pallas as pl


def kernel(x, z, w0, w1, gamma1, beta1, w2, gamma2, beta2, w3, gamma3, beta3, w4, bias4, gamma4, beta4, z1_w, z2_w, z2_b, xz1_w, xz2_w, xz3_w, xz3_b, f_w, f_b):
    raise NotImplementedError("write your pallas kernel here")


# speedup vs baseline: 1.1314x; 1.1314x over previous
import jax
import jax.numpy as jnp
from jax.experimental import pallas as pl
from jax.experimental.pallas import tpu as pltpu

_EPS = 1e-5


def _im2col_bf16(h, k, s, p):
    B, H, W, C = h.shape
    hp = jnp.pad(h, ((0, 0), (p, p), (p, p), (0, 0)))
    OH = (H + 2 * p - k) // s + 1
    OW = (W + 2 * p - k) // s + 1
    cols = []
    for kh in range(k):
        for kw in range(k):
            cols.append(hp[:, kh:kh + s * OH:s, kw:kw + s * OW:s, :])
    patches = jnp.stack(cols, axis=3)
    return patches.reshape(B * OH * OW, k * k * C), OH, OW


def _wmat(w):
    cout = w.shape[0]
    return w.transpose(2, 3, 1, 0).reshape(-1, cout).astype(jnp.bfloat16)


def _gemm_lrelu_body(a_ref, w_ref, o_ref):
    acc = jnp.dot(a_ref[...], w_ref[...], preferred_element_type=jnp.float32)
    o_ref[...] = jnp.maximum(acc, 0.2 * acc).astype(o_ref.dtype)


def _gemm_lrelu(a, w, tm):
    M, K = a.shape
    N = w.shape[1]
    return pl.pallas_call(
        _gemm_lrelu_body,
        out_shape=jax.ShapeDtypeStruct((M, N), jnp.bfloat16),
        grid=(M // tm,),
        in_specs=[pl.BlockSpec((tm, K), lambda i: (i, 0)),
                  pl.BlockSpec((K, N), lambda i: (0, 0))],
        out_specs=pl.BlockSpec((tm, N), lambda i: (i, 0)),
        compiler_params=pltpu.CompilerParams(
            dimension_semantics=("parallel",)),
    )(a, w)


def _gemm_stats_body(a_ref, w_ref, y_ref, s_ref, q_ref):
    acc = jnp.dot(a_ref[...], w_ref[...], preferred_element_type=jnp.float32)
    y_ref[...] = acc
    s_ref[...] = jnp.sum(acc, axis=0, keepdims=True)[None]
    q_ref[...] = jnp.sum(acc * acc, axis=0, keepdims=True)[None]


def _gemm_stats_bias_body(a_ref, w_ref, b_ref, y_ref, s_ref, q_ref):
    acc = jnp.dot(a_ref[...], w_ref[...], preferred_element_type=jnp.float32)
    acc = acc + b_ref[...]
    y_ref[...] = acc
    s_ref[...] = jnp.sum(acc, axis=0, keepdims=True)[None]
    q_ref[...] = jnp.sum(acc * acc, axis=0, keepdims=True)[None]


def _gemm_stats(a, w, bias, tm):
    M, K = a.shape
    N = w.shape[1]
    g = M // tm
    in_specs = [pl.BlockSpec((tm, K), lambda i: (i, 0)),
                pl.BlockSpec((K, N), lambda i: (0, 0))]
    args = [a, w]
    body = _gemm_stats_body
    if bias is not None:
        in_specs.append(pl.BlockSpec((1, N), lambda i: (0, 0)))
        args.append(bias.astype(jnp.float32).reshape(1, N))
        body = _gemm_stats_bias_body
    y, s, q = pl.pallas_call(
        body,
        out_shape=(jax.ShapeDtypeStruct((M, N), jnp.float32),
                   jax.ShapeDtypeStruct((g, 1, N), jnp.float32),
                   jax.ShapeDtypeStruct((g, 1, N), jnp.float32)),
        grid=(g,),
        in_specs=in_specs,
        out_specs=(pl.BlockSpec((tm, N), lambda i: (i, 0)),
                   pl.BlockSpec((1, 1, N), lambda i: (i, 0, 0)),
                   pl.BlockSpec((1, 1, N), lambda i: (i, 0, 0))),
        compiler_params=pltpu.CompilerParams(
            dimension_semantics=("parallel",)),
    )(*args)
    return y, s, q


def _bn_lrelu_body(y_ref, s_ref, b_ref, o_ref):
    v = y_ref[...] * s_ref[...] + b_ref[...]
    o_ref[...] = jnp.maximum(v, 0.2 * v).astype(o_ref.dtype)


def _bn_lrelu(y, scale, shift, tm):
    M, N = y.shape
    return pl.pallas_call(
        _bn_lrelu_body,
        out_shape=jax.ShapeDtypeStruct((M, N), jnp.bfloat16),
        grid=(M // tm,),
        in_specs=[pl.BlockSpec((tm, N), lambda i: (i, 0)),
                  pl.BlockSpec((1, N), lambda i: (0, 0)),
                  pl.BlockSpec((1, N), lambda i: (0, 0))],
        out_specs=pl.BlockSpec((tm, N), lambda i: (i, 0)),
        compiler_params=pltpu.CompilerParams(
            dimension_semantics=("parallel",)),
    )(y, scale.reshape(1, N), shift.reshape(1, N))


def _conv_bn_layer(h, w, gamma, beta, bias, k, s, p, tm_gemm, tm_bn):
    B = h.shape[0]
    cout = w.shape[0]
    a, OH, OW = _im2col_bf16(h, k, s, p)
    y, ps, pq = _gemm_stats(a, _wmat(w), bias, tm_gemm)
    M = a.shape[0]
    mean = ps.sum(axis=(0, 1)) / M
    var = pq.sum(axis=(0, 1)) / M - mean * mean
    scale = gamma * jax.lax.rsqrt(var + _EPS)
    shift = beta - mean * scale
    rows = _bn_lrelu(y, scale, shift, tm_bn)
    return rows.reshape(B, OH, OW, cout)


def _z_body(z_ref, w1_ref, w2_ref, b2_ref, wz_ref, o_ref):
    h = jnp.dot(z_ref[...], w1_ref[...], preferred_element_type=jnp.float32)
    h = jnp.maximum(h, 0.2 * h).astype(jnp.bfloat16)
    h = jnp.dot(h, w2_ref[...], preferred_element_type=jnp.float32)
    h = h + b2_ref[...]
    h = jnp.maximum(h, 0.2 * h).astype(jnp.bfloat16)
    o_ref[...] = jnp.dot(h, wz_ref[...], preferred_element_type=jnp.float32)


def _z_stack(z_rows, z1_w, z2_w, z2_b, wz):
    B = z_rows.shape[0]
    N = wz.shape[1]
    return pl.pallas_call(
        _z_body,
        out_shape=jax.ShapeDtypeStruct((B, N), jnp.float32),
    )(z_rows.astype(jnp.bfloat16), z1_w.astype(jnp.bfloat16),
      z2_w.astype(jnp.bfloat16), z2_b.astype(jnp.float32).reshape(1, -1),
      wz.astype(jnp.bfloat16))


def _xz_body(x_ref, zt_ref, wx_ref, w2_ref, w3_ref, b3_ref, fw_ref, fb_ref,
             o_ref):
    nb, nz = zt_ref.shape
    hw = x_ref.shape[0] // nb
    zb = jnp.broadcast_to(zt_ref[...][:, None, :], (nb, hw, nz))
    zb = zb.reshape(nb * hw, nz)
    h = jnp.dot(x_ref[...], wx_ref[...], preferred_element_type=jnp.float32)
    h = h + zb
    h = jnp.maximum(h, 0.2 * h).astype(jnp.bfloat16)
    h = jnp.dot(h, w2_ref[...], preferred_element_type=jnp.float32)
    h = jnp.maximum(h, 0.2 * h).astype(jnp.bfloat16)
    h = jnp.dot(h, w3_ref[...], preferred_element_type=jnp.float32)
    h = h + b3_ref[...]
    h = jnp.maximum(h, 0.2 * h)
    fd = h.shape[1]
    pooled = jnp.mean(h.reshape(nb, hw, fd), axis=1)
    logit = jnp.dot(pooled.astype(jnp.bfloat16), fw_ref[...],
                    preferred_element_type=jnp.float32) + fb_ref[...]
    o_ref[...] = jax.nn.sigmoid(logit)


def _xz_stack(x_rows, zterm, wx, w2, w3, b3, fw, fb, batch_blk=16, hw=16):
    BHW, fd = x_rows.shape
    n2 = wx.shape[1]
    B = BHW // hw
    g = B // batch_blk
    tm = batch_blk * hw
    fw_p = jnp.pad(fw.astype(jnp.bfloat16), ((0, 0), (0, 128 - fw.shape[1])))
    fb_p = jnp.broadcast_to(fb.astype(jnp.float32).reshape(1, -1), (1, 128))
    out = pl.pallas_call(
        _xz_body,
        out_shape=jax.ShapeDtypeStruct((B, 128), jnp.float32),
        grid=(g,),
        in_specs=[pl.BlockSpec((tm, fd), lambda i: (i, 0)),
                  pl.BlockSpec((batch_blk, n2), lambda i: (i, 0)),
                  pl.BlockSpec((fd, n2), lambda i: (0, 0)),
                  pl.BlockSpec((n2, fd), lambda i: (0, 0)),
                  pl.BlockSpec((fd, fd), lambda i: (0, 0)),
                  pl.BlockSpec((1, fd), lambda i: (0, 0)),
                  pl.BlockSpec((fd, 128), lambda i: (0, 0)),
                  pl.BlockSpec((1, 128), lambda i: (0, 0))],
        out_specs=pl.BlockSpec((batch_blk, 128), lambda i: (i, 0)),
        compiler_params=pltpu.CompilerParams(
            dimension_semantics=("parallel",)),
    )(x_rows, zterm, wx.astype(jnp.bfloat16), w2.astype(jnp.bfloat16),
      w3.astype(jnp.bfloat16), b3.astype(jnp.float32).reshape(1, fd),
      fw_p, fb_p)
    return out[:, :1]


def kernel(x, z, w0, w1, gamma1, beta1, w2, gamma2, beta2, w3, gamma3, beta3,
           w4, bias4, gamma4, beta4, z1_w, z2_w, z2_b, xz1_w, xz2_w,
           xz3_w, xz3_b, f_w, f_b):
    B = x.shape[0]
    fd = 512

    h = x.transpose(0, 2, 3, 1).astype(jnp.bfloat16)

    a0, OH, OW = _im2col_bf16(h, 4, 2, 1)
    r0 = _gemm_lrelu(a0, _wmat(w0), tm=4096)
    h = r0.reshape(B, OH, OW, w0.shape[0])

    h = _conv_bn_layer(h, w1, gamma1, beta1, None, 4, 2, 1, 2048, 4096)
    h = _conv_bn_layer(h, w2, gamma2, beta2, None, 4, 2, 1, 1024, 2048)
    h = _conv_bn_layer(h, w3, gamma3, beta3, None, 4, 2, 1, 512, 512)
    h = _conv_bn_layer(h, w4, gamma4, beta4, bias4, 3, 1, 1, 512, 512)

    HW = h.shape[1] * h.shape[2]
    x_rows = h.reshape(B * HW, fd)

    zterm = _z_stack(z.reshape(B, -1), z1_w, z2_w, z2_b, xz1_w[fd:])

    return _xz_stack(x_rows, zterm, xz1_w[:fd], xz2_w, xz3_w, xz3_b,
                     f_w, f_b, batch_blk=16, hw=HW)


# speedup vs baseline: 11.7465x; 10.3824x over previous
import jax
import jax.numpy as jnp
from jax.experimental import pallas as pl
from jax.experimental.pallas import tpu as pltpu

_EPS = 1e-5


def _im2col_bf16(h, k, s, p):
    B, H, W, C = h.shape
    hp = jnp.pad(h, ((0, 0), (p, p), (p, p), (0, 0)))
    OH = (H + 2 * p - k) // s + 1
    OW = (W + 2 * p - k) // s + 1
    if s == 2 and k == 4:
        hp2 = hp.reshape(B, H + 2 * p, (W + 2 * p) // 2, 2 * C)
        cols = []
        for kh in range(k):
            for j in range(2):
                cols.append(hp2[:, kh:kh + 2 * OH:2, j:j + OW, :])
        patches = jnp.stack(cols, axis=3)
        return patches.reshape(B * OH * OW, k * k * C), OH, OW
    cols = []
    for kh in range(k):
        for kw in range(k):
            cols.append(hp[:, kh:kh + s * OH:s, kw:kw + s * OW:s, :])
    patches = jnp.stack(cols, axis=3)
    return patches.reshape(B * OH * OW, k * k * C), OH, OW


def _wmat(w):
    cout = w.shape[0]
    return w.transpose(2, 3, 1, 0).reshape(-1, cout).astype(jnp.bfloat16)


def _gemm_lrelu_body(a_ref, w_ref, o_ref):
    acc = jnp.dot(a_ref[...], w_ref[...], preferred_element_type=jnp.float32)
    o_ref[...] = jnp.maximum(acc, 0.2 * acc).astype(o_ref.dtype)


def _gemm_lrelu(a, w, tm):
    M, K = a.shape
    N = w.shape[1]
    return pl.pallas_call(
        _gemm_lrelu_body,
        out_shape=jax.ShapeDtypeStruct((M, N), jnp.bfloat16),
        grid=(M // tm,),
        in_specs=[pl.BlockSpec((tm, K), lambda i: (i, 0)),
                  pl.BlockSpec((K, N), lambda i: (0, 0))],
        out_specs=pl.BlockSpec((tm, N), lambda i: (i, 0)),
        compiler_params=pltpu.CompilerParams(
            dimension_semantics=("parallel",)),
    )(a, w)


def _gemm_stats_body(a_ref, w_ref, y_ref, s_ref, q_ref):
    acc = jnp.dot(a_ref[...], w_ref[...], preferred_element_type=jnp.float32)
    y_ref[...] = acc
    s_ref[...] = jnp.sum(acc, axis=0, keepdims=True)[None]
    q_ref[...] = jnp.sum(acc * acc, axis=0, keepdims=True)[None]


def _gemm_stats_bias_body(a_ref, w_ref, b_ref, y_ref, s_ref, q_ref):
    acc = jnp.dot(a_ref[...], w_ref[...], preferred_element_type=jnp.float32)
    acc = acc + b_ref[...]
    y_ref[...] = acc
    s_ref[...] = jnp.sum(acc, axis=0, keepdims=True)[None]
    q_ref[...] = jnp.sum(acc * acc, axis=0, keepdims=True)[None]


def _gemm_stats(a, w, bias, tm):
    M, K = a.shape
    N = w.shape[1]
    g = M // tm
    in_specs = [pl.BlockSpec((tm, K), lambda i: (i, 0)),
                pl.BlockSpec((K, N), lambda i: (0, 0))]
    args = [a, w]
    body = _gemm_stats_body
    if bias is not None:
        in_specs.append(pl.BlockSpec((1, N), lambda i: (0, 0)))
        args.append(bias.astype(jnp.float32).reshape(1, N))
        body = _gemm_stats_bias_body
    y, s, q = pl.pallas_call(
        body,
        out_shape=(jax.ShapeDtypeStruct((M, N), jnp.float32),
                   jax.ShapeDtypeStruct((g, 1, N), jnp.float32),
                   jax.ShapeDtypeStruct((g, 1, N), jnp.float32)),
        grid=(g,),
        in_specs=in_specs,
        out_specs=(pl.BlockSpec((tm, N), lambda i: (i, 0)),
                   pl.BlockSpec((1, 1, N), lambda i: (i, 0, 0)),
                   pl.BlockSpec((1, 1, N), lambda i: (i, 0, 0))),
        compiler_params=pltpu.CompilerParams(
            dimension_semantics=("parallel",)),
    )(*args)
    return y, s, q


def _bn_lrelu_body(y_ref, s_ref, b_ref, o_ref):
    v = y_ref[...] * s_ref[...] + b_ref[...]
    o_ref[...] = jnp.maximum(v, 0.2 * v).astype(o_ref.dtype)


def _bn_lrelu(y, scale, shift, tm):
    M, N = y.shape
    return pl.pallas_call(
        _bn_lrelu_body,
        out_shape=jax.ShapeDtypeStruct((M, N), jnp.bfloat16),
        grid=(M // tm,),
        in_specs=[pl.BlockSpec((tm, N), lambda i: (i, 0)),
                  pl.BlockSpec((1, N), lambda i: (0, 0)),
                  pl.BlockSpec((1, N), lambda i: (0, 0))],
        out_specs=pl.BlockSpec((tm, N), lambda i: (i, 0)),
        compiler_params=pltpu.CompilerParams(
            dimension_semantics=("parallel",)),
    )(y, scale.reshape(1, N), shift.reshape(1, N))


def _conv_bn_layer(h, w, gamma, beta, bias, k, s, p, tm_gemm, tm_bn):
    B = h.shape[0]
    cout = w.shape[0]
    a, OH, OW = _im2col_bf16(h, k, s, p)
    y, ps, pq = _gemm_stats(a, _wmat(w), bias, tm_gemm)
    M = a.shape[0]
    mean = ps.sum(axis=(0, 1)) / M
    var = pq.sum(axis=(0, 1)) / M - mean * mean
    scale = gamma * jax.lax.rsqrt(var + _EPS)
    shift = beta - mean * scale
    rows = _bn_lrelu(y, scale, shift, tm_bn)
    return rows.reshape(B, OH, OW, cout)


def _z_body(z_ref, w1_ref, w2_ref, b2_ref, wz_ref, o_ref):
    h = jnp.dot(z_ref[...], w1_ref[...], preferred_element_type=jnp.float32)
    h = jnp.maximum(h, 0.2 * h).astype(jnp.bfloat16)
    h = jnp.dot(h, w2_ref[...], preferred_element_type=jnp.float32)
    h = h + b2_ref[...]
    h = jnp.maximum(h, 0.2 * h).astype(jnp.bfloat16)
    o_ref[...] = jnp.dot(h, wz_ref[...], preferred_element_type=jnp.float32)


def _z_stack(z_rows, z1_w, z2_w, z2_b, wz):
    B = z_rows.shape[0]
    N = wz.shape[1]
    return pl.pallas_call(
        _z_body,
        out_shape=jax.ShapeDtypeStruct((B, N), jnp.float32),
    )(z_rows.astype(jnp.bfloat16), z1_w.astype(jnp.bfloat16),
      z2_w.astype(jnp.bfloat16), z2_b.astype(jnp.float32).reshape(1, -1),
      wz.astype(jnp.bfloat16))


def _xz_body(x_ref, zt_ref, wx_ref, w2_ref, w3_ref, b3_ref, fw_ref, fb_ref,
             o_ref):
    nb, nz = zt_ref.shape
    hw = x_ref.shape[0] // nb
    zb = jnp.broadcast_to(zt_ref[...][:, None, :], (nb, hw, nz))
    zb = zb.reshape(nb * hw, nz)
    h = jnp.dot(x_ref[...], wx_ref[...], preferred_element_type=jnp.float32)
    h = h + zb
    h = jnp.maximum(h, 0.2 * h).astype(jnp.bfloat16)
    h = jnp.dot(h, w2_ref[...], preferred_element_type=jnp.float32)
    h = jnp.maximum(h, 0.2 * h).astype(jnp.bfloat16)
    h = jnp.dot(h, w3_ref[...], preferred_element_type=jnp.float32)
    h = h + b3_ref[...]
    h = jnp.maximum(h, 0.2 * h)
    fd = h.shape[1]
    pooled = jnp.mean(h.reshape(nb, hw, fd), axis=1)
    logit = jnp.dot(pooled.astype(jnp.bfloat16), fw_ref[...],
                    preferred_element_type=jnp.float32) + fb_ref[...]
    o_ref[...] = jax.nn.sigmoid(logit)


def _xz_stack(x_rows, zterm, wx, w2, w3, b3, fw, fb, batch_blk=16, hw=16):
    BHW, fd = x_rows.shape
    n2 = wx.shape[1]
    B = BHW // hw
    g = B // batch_blk
    tm = batch_blk * hw
    fw_p = jnp.pad(fw.astype(jnp.bfloat16), ((0, 0), (0, 128 - fw.shape[1])))
    fb_p = jnp.broadcast_to(fb.astype(jnp.float32).reshape(1, -1), (1, 128))
    out = pl.pallas_call(
        _xz_body,
        out_shape=jax.ShapeDtypeStruct((B, 128), jnp.float32),
        grid=(g,),
        in_specs=[pl.BlockSpec((tm, fd), lambda i: (i, 0)),
                  pl.BlockSpec((batch_blk, n2), lambda i: (i, 0)),
                  pl.BlockSpec((fd, n2), lambda i: (0, 0)),
                  pl.BlockSpec((n2, fd), lambda i: (0, 0)),
                  pl.BlockSpec((fd, fd), lambda i: (0, 0)),
                  pl.BlockSpec((1, fd), lambda i: (0, 0)),
                  pl.BlockSpec((fd, 128), lambda i: (0, 0)),
                  pl.BlockSpec((1, 128), lambda i: (0, 0))],
        out_specs=pl.BlockSpec((batch_blk, 128), lambda i: (i, 0)),
        compiler_params=pltpu.CompilerParams(
            dimension_semantics=("parallel",)),
    )(x_rows, zterm, wx.astype(jnp.bfloat16), w2.astype(jnp.bfloat16),
      w3.astype(jnp.bfloat16), b3.astype(jnp.float32).reshape(1, fd),
      fw_p, fb_p)
    return out[:, :1]


def kernel(x, z, w0, w1, gamma1, beta1, w2, gamma2, beta2, w3, gamma3, beta3,
           w4, bias4, gamma4, beta4, z1_w, z2_w, z2_b, xz1_w, xz2_w,
           xz3_w, xz3_b, f_w, f_b):
    B = x.shape[0]
    fd = 512

    h = x.transpose(0, 2, 3, 1).astype(jnp.bfloat16)

    a0, OH, OW = _im2col_bf16(h, 4, 2, 1)
    r0 = _gemm_lrelu(a0, _wmat(w0), tm=4096)
    h = r0.reshape(B, OH, OW, w0.shape[0])

    h = _conv_bn_layer(h, w1, gamma1, beta1, None, 4, 2, 1, 2048, 4096)
    h = _conv_bn_layer(h, w2, gamma2, beta2, None, 4, 2, 1, 1024, 2048)
    h = _conv_bn_layer(h, w3, gamma3, beta3, None, 4, 2, 1, 512, 512)
    h = _conv_bn_layer(h, w4, gamma4, beta4, bias4, 3, 1, 1, 512, 512)

    HW = h.shape[1] * h.shape[2]
    x_rows = h.reshape(B * HW, fd)

    zterm = _z_stack(z.reshape(B, -1), z1_w, z2_w, z2_b, xz1_w[fd:])

    return _xz_stack(x_rows, zterm, xz1_w[:fd], xz2_w, xz3_w, xz3_b,
                     f_w, f_b, batch_blk=16, hw=HW)


# speedup vs baseline: 18.0194x; 1.5340x over previous
import jax
import jax.numpy as jnp
from jax.experimental import pallas as pl
from jax.experimental.pallas import tpu as pltpu

_EPS = 1e-5


def _im2col_bf16(h, k, s, p):
    B, H, W, C = h.shape
    hp = jnp.pad(h, ((0, 0), (p, p), (p, p), (0, 0)))
    OH = (H + 2 * p - k) // s + 1
    OW = (W + 2 * p - k) // s + 1
    if s == 2 and k == 4:
        hp2 = hp.reshape(B, H + 2 * p, (W + 2 * p) // 2, 2 * C)
        cols = []
        for kh in range(k):
            for j in range(2):
                cols.append(hp2[:, kh:kh + 2 * OH:2, j:j + OW, :])
        patches = jnp.stack(cols, axis=3)
        return patches.reshape(B * OH * OW, k * k * C), OH, OW
    cols = []
    for kh in range(k):
        for kw in range(k):
            cols.append(hp[:, kh:kh + s * OH:s, kw:kw + s * OW:s, :])
    patches = jnp.stack(cols, axis=3)
    return patches.reshape(B * OH * OW, k * k * C), OH, OW


def _wmat(w):
    cout = w.shape[0]
    return w.transpose(2, 3, 1, 0).reshape(-1, cout).astype(jnp.bfloat16)


def _gemm_lrelu_body(a_ref, w_ref, o_ref):
    acc = jnp.dot(a_ref[...], w_ref[...], preferred_element_type=jnp.float32)
    o_ref[...] = jnp.maximum(acc, 0.2 * acc).astype(o_ref.dtype)


def _gemm_lrelu(a, w, tm):
    M, K = a.shape
    N = w.shape[1]
    return pl.pallas_call(
        _gemm_lrelu_body,
        out_shape=jax.ShapeDtypeStruct((M, N), jnp.bfloat16),
        grid=(M // tm,),
        in_specs=[pl.BlockSpec((tm, K), lambda i: (i, 0)),
                  pl.BlockSpec((K, N), lambda i: (0, 0))],
        out_specs=pl.BlockSpec((tm, N), lambda i: (i, 0)),
        compiler_params=pltpu.CompilerParams(
            dimension_semantics=("parallel",)),
    )(a, w)


def _make_conv_s2_body(G, OH, OW, C2):
    def body(x_ref, w_ref, y_ref, s_ref, q_ref):
        v = x_ref[...]
        acc = None
        for j in range(2):
            Sj = v[:, :, j:j + OW, :]
            P2 = Sj.reshape(G, OH + 1, 2, OW, C2)
            for kh in range(4):
                par, off = kh % 2, kh // 2
                a = P2[:, off:off + OH, par].reshape(G * OH * OW, C2)
                wp = w_ref[(2 * kh + j) * C2:(2 * kh + j + 1) * C2, :]
                d = jnp.dot(a, wp, preferred_element_type=jnp.float32)
                acc = d if acc is None else acc + d
        y_ref[...] = acc
        s_ref[...] = jnp.sum(acc, axis=0, keepdims=True)[None]
        q_ref[...] = jnp.sum(acc * acc, axis=0, keepdims=True)[None]
    return body


def _make_conv_s1_body(G, OH, OW, C, has_bias):
    def body(x_ref, w_ref, *rest):
        if has_bias:
            b_ref, y_ref, s_ref, q_ref = rest
        else:
            y_ref, s_ref, q_ref = rest
        v = x_ref[...]
        acc = None
        for j in range(3):
            Sj = v[:, :, j:j + OW, :]
            for kh in range(3):
                a = Sj[:, kh:kh + OH].reshape(G * OH * OW, C)
                wp = w_ref[(kh * 3 + j) * C:(kh * 3 + j + 1) * C, :]
                d = jnp.dot(a, wp, preferred_element_type=jnp.float32)
                acc = d if acc is None else acc + d
        if has_bias:
            acc = acc + b_ref[...]
        y_ref[...] = acc
        s_ref[...] = jnp.sum(acc, axis=0, keepdims=True)[None]
        q_ref[...] = jnp.sum(acc * acc, axis=0, keepdims=True)[None]
    return body


def _conv_stats(h, w, bias, k, s, G):
    B, H, W, C = h.shape
    N = w.shape[0]
    OH = (H + 2 - k) // s + 1
    OW = (W + 2 - k) // s + 1
    hp = jnp.pad(h, ((0, 0), (1, 1), (1, 1), (0, 0)))
    grid = B // G
    mloc = G * OH * OW
    M = B * OH * OW
    if s == 2:
        C2 = 2 * C
        xin = hp.reshape(B, H + 2, (W + 2) // 2, C2)
        body = _make_conv_s2_body(G, OH, OW, C2)
        xspec = pl.BlockSpec((G, H + 2, (W + 2) // 2, C2),
                             lambda i: (i, 0, 0, 0))
        args = [xin, _wmat(w)]
    else:
        body = _make_conv_s1_body(G, OH, OW, C, bias is not None)
        xspec = pl.BlockSpec((G, H + 2, W + 2, C), lambda i: (i, 0, 0, 0))
        args = [hp, _wmat(w)]
    in_specs = [xspec, pl.BlockSpec((k * k * C, N), lambda i: (0, 0))]
    if bias is not None:
        in_specs.append(pl.BlockSpec((1, N), lambda i: (0, 0)))
        args.append(bias.astype(jnp.float32).reshape(1, N))
    y, ps, pq = pl.pallas_call(
        body,
        out_shape=(jax.ShapeDtypeStruct((M, N), jnp.float32),
                   jax.ShapeDtypeStruct((grid, 1, N), jnp.float32),
                   jax.ShapeDtypeStruct((grid, 1, N), jnp.float32)),
        grid=(grid,),
        in_specs=in_specs,
        out_specs=(pl.BlockSpec((mloc, N), lambda i: (i, 0)),
                   pl.BlockSpec((1, 1, N), lambda i: (i, 0, 0)),
                   pl.BlockSpec((1, 1, N), lambda i: (i, 0, 0))),
        compiler_params=pltpu.CompilerParams(
            dimension_semantics=("parallel",)),
    )(*args)
    return y, ps, pq, OH, OW


def _bn_lrelu_body(y_ref, s_ref, b_ref, o_ref):
    v = y_ref[...] * s_ref[...] + b_ref[...]
    o_ref[...] = jnp.maximum(v, 0.2 * v).astype(o_ref.dtype)


def _bn_lrelu(y, scale, shift, tm):
    M, N = y.shape
    return pl.pallas_call(
        _bn_lrelu_body,
        out_shape=jax.ShapeDtypeStruct((M, N), jnp.bfloat16),
        grid=(M // tm,),
        in_specs=[pl.BlockSpec((tm, N), lambda i: (i, 0)),
                  pl.BlockSpec((1, N), lambda i: (0, 0)),
                  pl.BlockSpec((1, N), lambda i: (0, 0))],
        out_specs=pl.BlockSpec((tm, N), lambda i: (i, 0)),
        compiler_params=pltpu.CompilerParams(
            dimension_semantics=("parallel",)),
    )(y, scale.reshape(1, N), shift.reshape(1, N))


def _conv_bn_layer(h, w, gamma, beta, bias, k, s, G, tm_bn):
    B = h.shape[0]
    cout = w.shape[0]
    y, ps, pq, OH, OW = _conv_stats(h, w, bias, k, s, G)
    M = B * OH * OW
    mean = ps.sum(axis=(0, 1)) / M
    var = pq.sum(axis=(0, 1)) / M - mean * mean
    scale = gamma * jax.lax.rsqrt(var + _EPS)
    shift = beta - mean * scale
    rows = _bn_lrelu(y, scale, shift, tm_bn)
    return rows.reshape(B, OH, OW, cout)


def _z_body(z_ref, w1_ref, w2_ref, b2_ref, wz_ref, o_ref):
    h = jnp.dot(z_ref[...], w1_ref[...], preferred_element_type=jnp.float32)
    h = jnp.maximum(h, 0.2 * h).astype(jnp.bfloat16)
    h = jnp.dot(h, w2_ref[...], preferred_element_type=jnp.float32)
    h = h + b2_ref[...]
    h = jnp.maximum(h, 0.2 * h).astype(jnp.bfloat16)
    o_ref[...] = jnp.dot(h, wz_ref[...], preferred_element_type=jnp.float32)


def _z_stack(z_rows, z1_w, z2_w, z2_b, wz):
    B = z_rows.shape[0]
    N = wz.shape[1]
    return pl.pallas_call(
        _z_body,
        out_shape=jax.ShapeDtypeStruct((B, N), jnp.float32),
    )(z_rows.astype(jnp.bfloat16), z1_w.astype(jnp.bfloat16),
      z2_w.astype(jnp.bfloat16), z2_b.astype(jnp.float32).reshape(1, -1),
      wz.astype(jnp.bfloat16))


def _xz_body(x_ref, zt_ref, wx_ref, w2_ref, w3_ref, b3_ref, fw_ref, fb_ref,
             o_ref):
    nb, nz = zt_ref.shape
    hw = x_ref.shape[0] // nb
    zb = jnp.broadcast_to(zt_ref[...][:, None, :], (nb, hw, nz))
    zb = zb.reshape(nb * hw, nz)
    h = jnp.dot(x_ref[...], wx_ref[...], preferred_element_type=jnp.float32)
    h = h + zb
    h = jnp.maximum(h, 0.2 * h).astype(jnp.bfloat16)
    h = jnp.dot(h, w2_ref[...], preferred_element_type=jnp.float32)
    h = jnp.maximum(h, 0.2 * h).astype(jnp.bfloat16)
    h = jnp.dot(h, w3_ref[...], preferred_element_type=jnp.float32)
    h = h + b3_ref[...]
    h = jnp.maximum(h, 0.2 * h)
    fd = h.shape[1]
    pooled = jnp.mean(h.reshape(nb, hw, fd), axis=1)
    logit = jnp.dot(pooled.astype(jnp.bfloat16), fw_ref[...],
                    preferred_element_type=jnp.float32) + fb_ref[...]
    o_ref[...] = jax.nn.sigmoid(logit)


def _xz_stack(x_rows, zterm, wx, w2, w3, b3, fw, fb, batch_blk=16, hw=16):
    BHW, fd = x_rows.shape
    n2 = wx.shape[1]
    B = BHW // hw
    g = B // batch_blk
    tm = batch_blk * hw
    fw_p = jnp.pad(fw.astype(jnp.bfloat16), ((0, 0), (0, 128 - fw.shape[1])))
    fb_p = jnp.broadcast_to(fb.astype(jnp.float32).reshape(1, -1), (1, 128))
    out = pl.pallas_call(
        _xz_body,
        out_shape=jax.ShapeDtypeStruct((B, 128), jnp.float32),
        grid=(g,),
        in_specs=[pl.BlockSpec((tm, fd), lambda i: (i, 0)),
                  pl.BlockSpec((batch_blk, n2), lambda i: (i, 0)),
                  pl.BlockSpec((fd, n2), lambda i: (0, 0)),
                  pl.BlockSpec((n2, fd), lambda i: (0, 0)),
                  pl.BlockSpec((fd, fd), lambda i: (0, 0)),
                  pl.BlockSpec((1, fd), lambda i: (0, 0)),
                  pl.BlockSpec((fd, 128), lambda i: (0, 0)),
                  pl.BlockSpec((1, 128), lambda i: (0, 0))],
        out_specs=pl.BlockSpec((batch_blk, 128), lambda i: (i, 0)),
        compiler_params=pltpu.CompilerParams(
            dimension_semantics=("parallel",)),
    )(x_rows, zterm, wx.astype(jnp.bfloat16), w2.astype(jnp.bfloat16),
      w3.astype(jnp.bfloat16), b3.astype(jnp.float32).reshape(1, fd),
      fw_p, fb_p)
    return out[:, :1]


def kernel(x, z, w0, w1, gamma1, beta1, w2, gamma2, beta2, w3, gamma3, beta3,
           w4, bias4, gamma4, beta4, z1_w, z2_w, z2_b, xz1_w, xz2_w,
           xz3_w, xz3_b, f_w, f_b):
    B = x.shape[0]
    fd = 512

    h = x.transpose(0, 2, 3, 1).astype(jnp.bfloat16)

    a0, OH, OW = _im2col_bf16(h, 4, 2, 1)
    r0 = _gemm_lrelu(a0, _wmat(w0), tm=4096)
    h = r0.reshape(B, OH, OW, w0.shape[0])

    h = _conv_bn_layer(h, w1, gamma1, beta1, None, 4, 2, 8, 4096)
    h = _conv_bn_layer(h, w2, gamma2, beta2, None, 4, 2, 16, 2048)
    h = _conv_bn_layer(h, w3, gamma3, beta3, None, 4, 2, 32, 512)
    h = _conv_bn_layer(h, w4, gamma4, beta4, bias4, 3, 1, 32, 512)

    HW = h.shape[1] * h.shape[2]
    x_rows = h.reshape(B * HW, fd)

    zterm = _z_stack(z.reshape(B, -1), z1_w, z2_w, z2_b, xz1_w[fd:])

    return _xz_stack(x_rows, zterm, xz1_w[:fd], xz2_w, xz3_w, xz3_b,
                     f_w, f_b, batch_blk=16, hw=HW)


# speedup vs baseline: 19.2698x; 1.0694x over previous
import jax
import jax.numpy as jnp
from jax.experimental import pallas as pl
from jax.experimental.pallas import tpu as pltpu

_EPS = 1e-5


def _wmat(w):
    cout = w.shape[0]
    return w.transpose(2, 3, 1, 0).reshape(-1, cout).astype(jnp.bfloat16)


def _make_conv_s2_lrelu_body(G, OH, OW, C2):
    def body(x_ref, w_ref, o_ref):
        v = x_ref[...]
        acc = None
        for j in range(2):
            Sj = v[:, :, j:j + OW, :]
            P2 = Sj.reshape(G, OH + 1, 2, OW, C2)
            for kh in range(4):
                par, off = kh % 2, kh // 2
                a = P2[:, off:off + OH, par].reshape(G * OH * OW, C2)
                wp = w_ref[(2 * kh + j) * C2:(2 * kh + j + 1) * C2, :]
                d = jnp.dot(a, wp, preferred_element_type=jnp.float32)
                acc = d if acc is None else acc + d
        o_ref[...] = jnp.maximum(acc, 0.2 * acc).astype(o_ref.dtype)
    return body


def _conv_s2_lrelu(h, w, G):
    B, H, W, C = h.shape
    N = w.shape[0]
    OH, OW = H // 2, W // 2
    C2 = 2 * C
    hp = jnp.pad(h, ((0, 0), (1, 1), (1, 1), (0, 0)))
    xin = hp.reshape(B, H + 2, (W + 2) // 2, C2)
    mloc = G * OH * OW
    out = pl.pallas_call(
        _make_conv_s2_lrelu_body(G, OH, OW, C2),
        out_shape=jax.ShapeDtypeStruct((B * OH * OW, N), jnp.bfloat16),
        grid=(B // G,),
        in_specs=[pl.BlockSpec((G, H + 2, (W + 2) // 2, C2),
                               lambda i: (i, 0, 0, 0)),
                  pl.BlockSpec((16 * C, N), lambda i: (0, 0))],
        out_specs=pl.BlockSpec((mloc, N), lambda i: (i, 0)),
        compiler_params=pltpu.CompilerParams(
            dimension_semantics=("parallel",)),
    )(xin, _wmat(w))
    return out.reshape(B, OH, OW, N)


def _make_conv_s2_body(G, OH, OW, C2):
    def body(x_ref, w_ref, y_ref, s_ref, q_ref):
        v = x_ref[...]
        acc = None
        for j in range(2):
            Sj = v[:, :, j:j + OW, :]
            P2 = Sj.reshape(G, OH + 1, 2, OW, C2)
            for kh in range(4):
                par, off = kh % 2, kh // 2
                a = P2[:, off:off + OH, par].reshape(G * OH * OW, C2)
                wp = w_ref[(2 * kh + j) * C2:(2 * kh + j + 1) * C2, :]
                d = jnp.dot(a, wp, preferred_element_type=jnp.float32)
                acc = d if acc is None else acc + d
        y_ref[...] = acc
        s_ref[...] = jnp.sum(acc, axis=0, keepdims=True)[None]
        q_ref[...] = jnp.sum(acc * acc, axis=0, keepdims=True)[None]
    return body


def _make_conv_s1_body(G, OH, OW, C, has_bias):
    def body(x_ref, w_ref, *rest):
        if has_bias:
            b_ref, y_ref, s_ref, q_ref = rest
        else:
            y_ref, s_ref, q_ref = rest
        v = x_ref[...]
        acc = None
        for j in range(3):
            Sj = v[:, :, j:j + OW, :]
            for kh in range(3):
                a = Sj[:, kh:kh + OH].reshape(G * OH * OW, C)
                wp = w_ref[(kh * 3 + j) * C:(kh * 3 + j + 1) * C, :]
                d = jnp.dot(a, wp, preferred_element_type=jnp.float32)
                acc = d if acc is None else acc + d
        if has_bias:
            acc = acc + b_ref[...]
        y_ref[...] = acc
        s_ref[...] = jnp.sum(acc, axis=0, keepdims=True)[None]
        q_ref[...] = jnp.sum(acc * acc, axis=0, keepdims=True)[None]
    return body


def _conv_stats(h, w, bias, k, s, G):
    B, H, W, C = h.shape
    N = w.shape[0]
    OH = (H + 2 - k) // s + 1
    OW = (W + 2 - k) // s + 1
    hp = jnp.pad(h, ((0, 0), (1, 1), (1, 1), (0, 0)))
    grid = B // G
    mloc = G * OH * OW
    M = B * OH * OW
    if s == 2:
        C2 = 2 * C
        xin = hp.reshape(B, H + 2, (W + 2) // 2, C2)
        body = _make_conv_s2_body(G, OH, OW, C2)
        xspec = pl.BlockSpec((G, H + 2, (W + 2) // 2, C2),
                             lambda i: (i, 0, 0, 0))
        args = [xin, _wmat(w)]
    else:
        body = _make_conv_s1_body(G, OH, OW, C, bias is not None)
        xspec = pl.BlockSpec((G, H + 2, W + 2, C), lambda i: (i, 0, 0, 0))
        args = [hp, _wmat(w)]
    in_specs = [xspec, pl.BlockSpec((k * k * C, N), lambda i: (0, 0))]
    if bias is not None:
        in_specs.append(pl.BlockSpec((1, N), lambda i: (0, 0)))
        args.append(bias.astype(jnp.float32).reshape(1, N))
    y, ps, pq = pl.pallas_call(
        body,
        out_shape=(jax.ShapeDtypeStruct((M, N), jnp.float32),
                   jax.ShapeDtypeStruct((grid, 1, N), jnp.float32),
                   jax.ShapeDtypeStruct((grid, 1, N), jnp.float32)),
        grid=(grid,),
        in_specs=in_specs,
        out_specs=(pl.BlockSpec((mloc, N), lambda i: (i, 0)),
                   pl.BlockSpec((1, 1, N), lambda i: (i, 0, 0)),
                   pl.BlockSpec((1, 1, N), lambda i: (i, 0, 0))),
        compiler_params=pltpu.CompilerParams(
            dimension_semantics=("parallel",)),
    )(*args)
    return y, ps, pq, OH, OW


def _bn_lrelu_body(y_ref, s_ref, b_ref, o_ref):
    v = y_ref[...] * s_ref[...] + b_ref[...]
    o_ref[...] = jnp.maximum(v, 0.2 * v).astype(o_ref.dtype)


def _bn_lrelu(y, scale, shift, tm):
    M, N = y.shape
    return pl.pallas_call(
        _bn_lrelu_body,
        out_shape=jax.ShapeDtypeStruct((M, N), jnp.bfloat16),
        grid=(M // tm,),
        in_specs=[pl.BlockSpec((tm, N), lambda i: (i, 0)),
                  pl.BlockSpec((1, N), lambda i: (0, 0)),
                  pl.BlockSpec((1, N), lambda i: (0, 0))],
        out_specs=pl.BlockSpec((tm, N), lambda i: (i, 0)),
        compiler_params=pltpu.CompilerParams(
            dimension_semantics=("parallel",)),
    )(y, scale.reshape(1, N), shift.reshape(1, N))


def _conv_bn_layer(h, w, gamma, beta, bias, k, s, G, tm_bn):
    B = h.shape[0]
    cout = w.shape[0]
    y, ps, pq, OH, OW = _conv_stats(h, w, bias, k, s, G)
    M = B * OH * OW
    mean = ps.sum(axis=(0, 1)) / M
    var = pq.sum(axis=(0, 1)) / M - mean * mean
    scale = gamma * jax.lax.rsqrt(var + _EPS)
    shift = beta - mean * scale
    rows = _bn_lrelu(y, scale, shift, tm_bn)
    return rows.reshape(B, OH, OW, cout)


def _z_body(z_ref, w1_ref, w2_ref, b2_ref, wz_ref, o_ref):
    h = jnp.dot(z_ref[...], w1_ref[...], preferred_element_type=jnp.float32)
    h = jnp.maximum(h, 0.2 * h).astype(jnp.bfloat16)
    h = jnp.dot(h, w2_ref[...], preferred_element_type=jnp.float32)
    h = h + b2_ref[...]
    h = jnp.maximum(h, 0.2 * h).astype(jnp.bfloat16)
    o_ref[...] = jnp.dot(h, wz_ref[...], preferred_element_type=jnp.float32)


def _z_stack(z_rows, z1_w, z2_w, z2_b, wz):
    B = z_rows.shape[0]
    N = wz.shape[1]
    return pl.pallas_call(
        _z_body,
        out_shape=jax.ShapeDtypeStruct((B, N), jnp.float32),
    )(z_rows.astype(jnp.bfloat16), z1_w.astype(jnp.bfloat16),
      z2_w.astype(jnp.bfloat16), z2_b.astype(jnp.float32).reshape(1, -1),
      wz.astype(jnp.bfloat16))


def _xz_body(x_ref, zt_ref, wx_ref, w2_ref, w3_ref, b3_ref, fw_ref, fb_ref,
             o_ref):
    nb, nz = zt_ref.shape
    hw = x_ref.shape[0] // nb
    zb = jnp.broadcast_to(zt_ref[...][:, None, :], (nb, hw, nz))
    zb = zb.reshape(nb * hw, nz)
    h = jnp.dot(x_ref[...], wx_ref[...], preferred_element_type=jnp.float32)
    h = h + zb
    h = jnp.maximum(h, 0.2 * h).astype(jnp.bfloat16)
    h = jnp.dot(h, w2_ref[...], preferred_element_type=jnp.float32)
    h = jnp.maximum(h, 0.2 * h).astype(jnp.bfloat16)
    h = jnp.dot(h, w3_ref[...], preferred_element_type=jnp.float32)
    h = h + b3_ref[...]
    h = jnp.maximum(h, 0.2 * h)
    fd = h.shape[1]
    pooled = jnp.mean(h.reshape(nb, hw, fd), axis=1)
    logit = jnp.dot(pooled.astype(jnp.bfloat16), fw_ref[...],
                    preferred_element_type=jnp.float32) + fb_ref[...]
    o_ref[...] = jax.nn.sigmoid(logit)


def _xz_stack(x_rows, zterm, wx, w2, w3, b3, fw, fb, batch_blk=16, hw=16):
    BHW, fd = x_rows.shape
    n2 = wx.shape[1]
    B = BHW // hw
    g = B // batch_blk
    tm = batch_blk * hw
    fw_p = jnp.pad(fw.astype(jnp.bfloat16), ((0, 0), (0, 128 - fw.shape[1])))
    fb_p = jnp.broadcast_to(fb.astype(jnp.float32).reshape(1, -1), (1, 128))
    out = pl.pallas_call(
        _xz_body,
        out_shape=jax.ShapeDtypeStruct((B, 128), jnp.float32),
        grid=(g,),
        in_specs=[pl.BlockSpec((tm, fd), lambda i: (i, 0)),
                  pl.BlockSpec((batch_blk, n2), lambda i: (i, 0)),
                  pl.BlockSpec((fd, n2), lambda i: (0, 0)),
                  pl.BlockSpec((n2, fd), lambda i: (0, 0)),
                  pl.BlockSpec((fd, fd), lambda i: (0, 0)),
                  pl.BlockSpec((1, fd), lambda i: (0, 0)),
                  pl.BlockSpec((fd, 128), lambda i: (0, 0)),
                  pl.BlockSpec((1, 128), lambda i: (0, 0))],
        out_specs=pl.BlockSpec((batch_blk, 128), lambda i: (i, 0)),
        compiler_params=pltpu.CompilerParams(
            dimension_semantics=("parallel",)),
    )(x_rows, zterm, wx.astype(jnp.bfloat16), w2.astype(jnp.bfloat16),
      w3.astype(jnp.bfloat16), b3.astype(jnp.float32).reshape(1, fd),
      fw_p, fb_p)
    return out[:, :1]


def kernel(x, z, w0, w1, gamma1, beta1, w2, gamma2, beta2, w3, gamma3, beta3,
           w4, bias4, gamma4, beta4, z1_w, z2_w, z2_b, xz1_w, xz2_w,
           xz3_w, xz3_b, f_w, f_b):
    B = x.shape[0]
    fd = 512

    h = x.transpose(0, 2, 3, 1).astype(jnp.bfloat16)

    h = _conv_s2_lrelu(h, w0, G=4)

    h = _conv_bn_layer(h, w1, gamma1, beta1, None, 4, 2, 8, 4096)
    h = _conv_bn_layer(h, w2, gamma2, beta2, None, 4, 2, 16, 2048)
    h = _conv_bn_layer(h, w3, gamma3, beta3, None, 4, 2, 32, 512)
    h = _conv_bn_layer(h, w4, gamma4, beta4, bias4, 3, 1, 32, 512)

    HW = h.shape[1] * h.shape[2]
    x_rows = h.reshape(B * HW, fd)

    zterm = _z_stack(z.reshape(B, -1), z1_w, z2_w, z2_b, xz1_w[fd:])

    return _xz_stack(x_rows, zterm, xz1_w[:fd], xz2_w, xz3_w, xz3_b,
                     f_w, f_b, batch_blk=16, hw=HW)
